# baseline (speedup 1.0000x reference)
import functools

import jax
import jax.numpy as jnp
from jax import lax
from jax.experimental import pallas as pl
from jax.experimental.pallas import tpu as pltpu
from jax.experimental.pallas import tpu_sc as plsc

N = 10000
E = 320000
F = 128
C = 128
NUM_ITER = 2

NS = 16
NW = 2 * NS
BIN = 320
CH = 64
IB = 16
EDGES = 2 * E
KG = 336
KBL = KG // IB
CAPB = KG * CH
ACC_ROWS = 328
TRASH = ACC_ROWS - 2
N_OUT = NW * BIN

BLK = 2000
GRID = N // BLK


def _init_body(obj_ref, wos_ref, wsa_ref, wlsa_ref, bs_ref, ba_ref,
               st_ref, m_ref, ea_ref):
    x = obj_ref[...]
    st = jnp.tanh(jnp.dot(x, wos_ref[...], preferred_element_type=jnp.float32)
                  + bs_ref[...])
    a = jnp.dot(st, wsa_ref[...], preferred_element_type=jnp.float32)
    l = jnp.dot(st, wlsa_ref[...], preferred_element_type=jnp.float32)
    p = jnp.exp(l)
    st_ref[...] = st
    m_ref[...] = jnp.concatenate([p, p * st], axis=1)
    ea_ref[...] = jnp.exp(a + ba_ref[...])


def _mid_body(st_ref, s_ref, t_ref, ea_ref, wls_ref, wsa_ref, wlsa_ref,
              bs_ref, ba_ref, nst_ref, m_ref, nea_ref):
    st = st_ref[...]
    ea = ea_ref[...]
    inv = 1.0 / (ea * s_ref[...] + 1.0)
    g = ea * inv * t_ref[...]
    nst = jnp.tanh(st * inv
                   + jnp.dot(g, wls_ref[...], preferred_element_type=jnp.float32)
                   + bs_ref[...])
    a = jnp.dot(nst, wsa_ref[...], preferred_element_type=jnp.float32)
    l = jnp.dot(nst, wlsa_ref[...], preferred_element_type=jnp.float32)
    p = jnp.exp(l)
    nst_ref[...] = nst
    m_ref[...] = jnp.concatenate([p, p * nst], axis=1)
    nea_ref[...] = jnp.exp(a + ba_ref[...])


def _final_body(st_ref, s_ref, t_ref, ea_ref, wls_ref, bs_ref, out_ref):
    st = st_ref[...]
    ea = ea_ref[...]
    inv = 1.0 / (ea * s_ref[...] + 1.0)
    g = ea * inv * t_ref[...]
    out_ref[...] = jnp.tanh(
        st * inv
        + jnp.dot(g, wls_ref[...], preferred_element_type=jnp.float32)
        + bs_ref[...])


_row_spec = pl.BlockSpec((BLK, C), lambda i: (i, 0))
_wide_spec = pl.BlockSpec((BLK, 2 * C), lambda i: (i, 0))
_w_spec = pl.BlockSpec((C, C), lambda i: (0, 0))
_b_spec = pl.BlockSpec((1, C), lambda i: (0, 0))
_nc_shape = jax.ShapeDtypeStruct((N, C), jnp.float32)
_wide_shape = jax.ShapeDtypeStruct((N, 2 * C), jnp.float32)


def _tc_init(obj, wos, wsa, wlsa, bs2, ba2):
    return pl.pallas_call(
        _init_body,
        grid=(GRID,),
        in_specs=[_row_spec, _w_spec, _w_spec, _w_spec, _b_spec, _b_spec],
        out_specs=[_row_spec, _wide_spec, _row_spec],
        out_shape=[_nc_shape, _wide_shape, _nc_shape],
    )(obj, wos, wsa, wlsa, bs2, ba2)


def _tc_mid(st, s, t, ea, wls, wsa, wlsa, bs2, ba2):
    return pl.pallas_call(
        _mid_body,
        grid=(GRID,),
        in_specs=[_row_spec] * 4 + [_w_spec] * 3 + [_b_spec] * 2,
        out_specs=[_row_spec, _wide_spec, _row_spec],
        out_shape=[_nc_shape, _wide_shape, _nc_shape],
    )(st, s, t, ea, wls, wsa, wlsa, bs2, ba2)


def _tc_final(st, s, t, ea, wls, bs2):
    return pl.pallas_call(
        _final_body,
        grid=(GRID,),
        in_specs=[_row_spec] * 4 + [_w_spec, _b_spec],
        out_specs=_row_spec,
        out_shape=_nc_shape,
    )(st, s, t, ea, wls, bs2)


@functools.lru_cache(maxsize=1)
def _build_segsum():
    @functools.partial(
        pl.kernel,
        out_type=jax.ShapeDtypeStruct((N_OUT, 2 * C), jnp.float32),
        mesh=plsc.VectorSubcoreMesh(core_axis_name="c", subcore_axis_name="s",
                                    num_cores=2, num_subcores=NS),
        scratch_types=[
            pltpu.VMEM((IB, CH), jnp.int32),
            pltpu.VMEM((IB, CH), jnp.int32),
            pltpu.VMEM((CH, 2 * C), jnp.float32),
            pltpu.VMEM((ACC_ROWS, 2 * C), jnp.float32),
            pltpu.SemaphoreType.DMA,
        ],
    )
    def _segsum(m_hbm, z_hbm, ct_hbm, cf_hbm, out,
                ct_v, cf_v, rows, accl, sem):
        cid = lax.axis_index("c")
        sid = lax.axis_index("s")
        w = cid * NS + sid
        pltpu.sync_copy(z_hbm, accl)

        def block(j, carry):
            pltpu.sync_copy(ct_hbm.at[w, pl.ds(j * IB, IB)], ct_v)
            pltpu.sync_copy(cf_hbm.at[w, pl.ds(j * IB, IB)], cf_v)

            def chunk(k, c2):
                pltpu.async_copy(m_hbm.at[ct_v.at[k]], rows, sem).wait()
                for g in range(CH // 16):
                    iv = cf_v[k, pl.ds(g * 16, 16)]
                    for l in range(16):
                        d = iv[l]
                        e = g * 16 + l
                        for j8 in range(16):
                            sl = pl.ds(j8 * 16, 16)
                            plsc.addupdate(accl.at[d, sl], rows[e, sl])
                return c2

            lax.fori_loop(0, IB, chunk, 0)
            return carry

        lax.fori_loop(0, KBL, block, 0)
        pltpu.sync_copy(accl.at[pl.ds(0, BIN)], out.at[pl.ds(w * BIN, BIN)])

    return _segsum


def kernel(objects, connections, object_state_W, state_attention_W,
           linked_state_attention_W, attention_b, linked_state_W, state_b):
    obj = objects[0]
    u = connections[0, :, 0]
    v = connections[0, :, 1]
    src = jnp.concatenate([v, u])
    dst = jnp.concatenate([u, v])
    b = dst // BIN
    oh = (b[:, None] == jnp.arange(NW, dtype=jnp.int32)[None, :]).astype(jnp.int32)
    rank = jnp.take_along_axis(jnp.cumsum(oh, axis=0), b[:, None], axis=1)[:, 0] - 1
    slot = b * CAPB + rank
    gidx = jnp.zeros((NW * CAPB,), jnp.int32).at[slot].set(src)
    lrow = dst - b * BIN
    sidx = jnp.full((NW * CAPB,), TRASH, jnp.int32).at[slot].set(lrow)
    ct_idx = gidx.reshape(NW, KG, CH)
    cf_idx = sidx.reshape(NW, KG, CH)
    zeros = jnp.zeros((ACC_ROWS, 2 * C), jnp.float32)

    bs2 = state_b.reshape(1, C)
    ba2 = attention_b.reshape(1, C)

    st, m, ea = _tc_init(obj, object_state_W, state_attention_W,
                         linked_state_attention_W, bs2, ba2)
    for it in range(NUM_ITER):
        o = _build_segsum()(m, zeros, ct_idx, cf_idx)
        s = o[:N, :C]
        t = o[:N, C:]
        if it < NUM_ITER - 1:
            st, m, ea = _tc_mid(st, s, t, ea, linked_state_W,
                                state_attention_W, linked_state_attention_W,
                                bs2, ba2)
        else:
            st = _tc_final(st, s, t, ea, linked_state_W, bs2)
    return st[None]


# speedup vs baseline: 4.5983x; 4.5983x over previous
import functools

import jax
import jax.numpy as jnp
from jax import lax
from jax.experimental import pallas as pl
from jax.experimental.pallas import tpu as pltpu
from jax.experimental.pallas import tpu_sc as plsc

N = 10000
E = 320000
F = 128
C = 128
NUM_ITER = 2

NS = 16
CHUNK = 128
IB = 16
EDGES = 2 * E
K = IB * (-(-EDGES // (NS * CHUNK * IB)))
KB = K // IB
EDGES_PAD = NS * K * CHUNK
ROWS_PER_TILE = 640
N_ACC = NS * ROWS_PER_TILE
TRASH_ROW = N_ACC - 2

BLK = 2000
GRID = N // BLK


def _init_body(obj_ref, wos_ref, wsa_ref, wlsa_ref, bs_ref, ba_ref,
               st_ref, p_ref, ps_ref, ea_ref):
    x = obj_ref[...]
    st = jnp.tanh(jnp.dot(x, wos_ref[...], preferred_element_type=jnp.float32)
                  + bs_ref[...])
    a = jnp.dot(st, wsa_ref[...], preferred_element_type=jnp.float32)
    l = jnp.dot(st, wlsa_ref[...], preferred_element_type=jnp.float32)
    p = jnp.exp(l)
    st_ref[...] = st
    p_ref[...] = p
    ps_ref[...] = p * st
    ea_ref[...] = jnp.exp(a + ba_ref[...])


def _mid_body(st_ref, s_ref, t_ref, ea_ref, wls_ref, wsa_ref, wlsa_ref,
              bs_ref, ba_ref, nst_ref, p_ref, ps_ref, nea_ref):
    st = st_ref[...]
    ea = ea_ref[...]
    inv = 1.0 / (ea * s_ref[...] + 1.0)
    g = ea * inv * t_ref[...]
    nst = jnp.tanh(st * inv
                   + jnp.dot(g, wls_ref[...], preferred_element_type=jnp.float32)
                   + bs_ref[...])
    a = jnp.dot(nst, wsa_ref[...], preferred_element_type=jnp.float32)
    l = jnp.dot(nst, wlsa_ref[...], preferred_element_type=jnp.float32)
    p = jnp.exp(l)
    nst_ref[...] = nst
    p_ref[...] = p
    ps_ref[...] = p * nst
    nea_ref[...] = jnp.exp(a + ba_ref[...])


def _final_body(st_ref, s_ref, t_ref, ea_ref, wls_ref, bs_ref, out_ref):
    st = st_ref[...]
    ea = ea_ref[...]
    inv = 1.0 / (ea * s_ref[...] + 1.0)
    g = ea * inv * t_ref[...]
    out_ref[...] = jnp.tanh(
        st * inv
        + jnp.dot(g, wls_ref[...], preferred_element_type=jnp.float32)
        + bs_ref[...])


_row_spec = pl.BlockSpec((BLK, C), lambda i: (i, 0))
_w_spec = pl.BlockSpec((C, C), lambda i: (0, 0))
_b_spec = pl.BlockSpec((1, C), lambda i: (0, 0))
_nc_shape = jax.ShapeDtypeStruct((N, C), jnp.float32)


def _tc_init(obj, wos, wsa, wlsa, bs2, ba2):
    return pl.pallas_call(
        _init_body,
        grid=(GRID,),
        in_specs=[_row_spec, _w_spec, _w_spec, _w_spec, _b_spec, _b_spec],
        out_specs=[_row_spec] * 4,
        out_shape=[_nc_shape] * 4,
    )(obj, wos, wsa, wlsa, bs2, ba2)


def _tc_mid(st, s, t, ea, wls, wsa, wlsa, bs2, ba2):
    return pl.pallas_call(
        _mid_body,
        grid=(GRID,),
        in_specs=[_row_spec] * 4 + [_w_spec] * 3 + [_b_spec] * 2,
        out_specs=[_row_spec] * 4,
        out_shape=[_nc_shape] * 4,
    )(st, s, t, ea, wls, wsa, wlsa, bs2, ba2)


def _tc_final(st, s, t, ea, wls, bs2):
    return pl.pallas_call(
        _final_body,
        grid=(GRID,),
        in_specs=[_row_spec] * 4 + [_w_spec, _b_spec],
        out_specs=_row_spec,
        out_shape=_nc_shape,
    )(st, s, t, ea, wls, bs2)


@functools.lru_cache(maxsize=1)
def _build_segsum():
    @functools.partial(
        pl.kernel,
        out_type=jax.ShapeDtypeStruct((2, N_ACC, C), jnp.float32),
        mesh=plsc.VectorSubcoreMesh(core_axis_name="c", subcore_axis_name="s",
                                    num_cores=2, num_subcores=NS),
        scratch_types=[
            pltpu.VMEM((IB, CHUNK), jnp.int32),
            pltpu.VMEM((IB, CHUNK), jnp.int32),
            pltpu.VMEM((CHUNK, C), jnp.float32),
            pltpu.VMEM((CHUNK, C), jnp.float32),
            pltpu.VMEM_SHARED((N_ACC, C), jnp.float32),
            pltpu.SemaphoreType.DMA,
        ],
    )
    def _segsum(p_hbm, ps_hbm, z_hbm, ct_hbm, cf_hbm, out,
                ct_v, cf_v, rows0, rows1, acc, sem):
        cid = lax.axis_index("c")
        sid = lax.axis_index("s")
        r0 = sid * ROWS_PER_TILE
        pltpu.sync_copy(z_hbm.at[pl.ds(r0, ROWS_PER_TILE)],
                        acc.at[pl.ds(r0, ROWS_PER_TILE)])
        plsc.subcore_barrier()

        bufs = (rows0, rows1)

        def run(tbl):
            def block(j, carry):
                pltpu.sync_copy(ct_hbm.at[sid, pl.ds(j * IB, IB)], ct_v)
                pltpu.sync_copy(cf_hbm.at[sid, pl.ds(j * IB, IB)], cf_v)
                desc = pltpu.async_copy(tbl.at[ct_v.at[0]], bufs[0], sem)
                for k in range(IB):
                    desc.wait()
                    if k + 1 < IB:
                        desc = pltpu.async_copy(tbl.at[ct_v.at[k + 1]],
                                                bufs[(k + 1) % 2], sem)
                    pltpu.sync_copy(bufs[k % 2], acc.at[cf_v.at[k]], add=True)
                return carry
            lax.fori_loop(0, KB, block, 0)
            plsc.subcore_barrier()
            pltpu.sync_copy(acc.at[pl.ds(r0, ROWS_PER_TILE)],
                            out.at[cid, pl.ds(r0, ROWS_PER_TILE)])

        @pl.when(cid == 0)
        def _():
            run(p_hbm)

        @pl.when(cid == 1)
        def _():
            run(ps_hbm)

    return _segsum


def kernel(objects, connections, object_state_W, state_attention_W,
           linked_state_attention_W, attention_b, linked_state_W, state_b):
    obj = objects[0]
    u = connections[0, :, 0]
    v = connections[0, :, 1]
    gat = jnp.concatenate([v, u])
    sca = jnp.concatenate([u, v])
    pad = EDGES_PAD - EDGES
    gat = jnp.concatenate([gat, jnp.zeros((pad,), jnp.int32)])
    sca = jnp.concatenate([sca, jnp.full((pad,), TRASH_ROW, jnp.int32)])
    ct_idx = gat.reshape(NS, K, CHUNK)
    cf_idx = sca.reshape(NS, K, CHUNK)
    zeros = jnp.zeros((N_ACC, C), jnp.float32)

    bs2 = state_b.reshape(1, C)
    ba2 = attention_b.reshape(1, C)

    st, p, ps, ea = _tc_init(obj, object_state_W, state_attention_W,
                             linked_state_attention_W, bs2, ba2)
    for it in range(NUM_ITER):
        o = _build_segsum()(p, ps, zeros, ct_idx, cf_idx)
        s_pad = o[0]
        t_pad = o[1]
        if it < NUM_ITER - 1:
            st, p, ps, ea = _tc_mid(st, s_pad, t_pad, ea, linked_state_W,
                                    state_attention_W, linked_state_attention_W,
                                    bs2, ba2)
        else:
            st = _tc_final(st, s_pad, t_pad, ea, linked_state_W, bs2)
    return st[None]


# speedup vs baseline: 4.6434x; 1.0098x over previous
import functools

import jax
import jax.numpy as jnp
from jax import lax
from jax.experimental import pallas as pl
from jax.experimental.pallas import tpu as pltpu
from jax.experimental.pallas import tpu_sc as plsc

N = 10000
E = 320000
F = 128
C = 128
NUM_ITER = 2

NS = 16
CHUNK = 128
IB = 32
EDGES = 2 * E
K = IB * (-(-EDGES // (NS * CHUNK * IB)))
KB = K // IB
EDGES_PAD = NS * K * CHUNK
ROWS_PER_TILE = 640
N_ACC = NS * ROWS_PER_TILE
TRASH_ROW = N_ACC - 2

BLK = 2000
GRID = N // BLK


def _init_body(obj_ref, wos_ref, wsa_ref, wlsa_ref, bs_ref, ba_ref,
               st_ref, p_ref, ps_ref, ea_ref):
    x = obj_ref[...]
    st = jnp.tanh(jnp.dot(x, wos_ref[...], preferred_element_type=jnp.float32)
                  + bs_ref[...])
    a = jnp.dot(st, wsa_ref[...], preferred_element_type=jnp.float32)
    l = jnp.dot(st, wlsa_ref[...], preferred_element_type=jnp.float32)
    p = jnp.exp(l)
    st_ref[...] = st
    p_ref[...] = p
    ps_ref[...] = p * st
    ea_ref[...] = jnp.exp(a + ba_ref[...])


def _mid_body(st_ref, s_ref, t_ref, ea_ref, wls_ref, wsa_ref, wlsa_ref,
              bs_ref, ba_ref, nst_ref, p_ref, ps_ref, nea_ref):
    st = st_ref[...]
    ea = ea_ref[...]
    inv = 1.0 / (ea * s_ref[...] + 1.0)
    g = ea * inv * t_ref[...]
    nst = jnp.tanh(st * inv
                   + jnp.dot(g, wls_ref[...], preferred_element_type=jnp.float32)
                   + bs_ref[...])
    a = jnp.dot(nst, wsa_ref[...], preferred_element_type=jnp.float32)
    l = jnp.dot(nst, wlsa_ref[...], preferred_element_type=jnp.float32)
    p = jnp.exp(l)
    nst_ref[...] = nst
    p_ref[...] = p
    ps_ref[...] = p * nst
    nea_ref[...] = jnp.exp(a + ba_ref[...])


def _final_body(st_ref, s_ref, t_ref, ea_ref, wls_ref, bs_ref, out_ref):
    st = st_ref[...]
    ea = ea_ref[...]
    inv = 1.0 / (ea * s_ref[...] + 1.0)
    g = ea * inv * t_ref[...]
    out_ref[...] = jnp.tanh(
        st * inv
        + jnp.dot(g, wls_ref[...], preferred_element_type=jnp.float32)
        + bs_ref[...])


_row_spec = pl.BlockSpec((BLK, C), lambda i: (i, 0))
_w_spec = pl.BlockSpec((C, C), lambda i: (0, 0))
_b_spec = pl.BlockSpec((1, C), lambda i: (0, 0))
_nc_shape = jax.ShapeDtypeStruct((N, C), jnp.float32)


def _tc_init(obj, wos, wsa, wlsa, bs2, ba2):
    return pl.pallas_call(
        _init_body,
        grid=(GRID,),
        in_specs=[_row_spec, _w_spec, _w_spec, _w_spec, _b_spec, _b_spec],
        out_specs=[_row_spec] * 4,
        out_shape=[_nc_shape] * 4,
    )(obj, wos, wsa, wlsa, bs2, ba2)


def _tc_mid(st, s, t, ea, wls, wsa, wlsa, bs2, ba2):
    return pl.pallas_call(
        _mid_body,
        grid=(GRID,),
        in_specs=[_row_spec] * 4 + [_w_spec] * 3 + [_b_spec] * 2,
        out_specs=[_row_spec] * 4,
        out_shape=[_nc_shape] * 4,
    )(st, s, t, ea, wls, wsa, wlsa, bs2, ba2)


def _tc_final(st, s, t, ea, wls, bs2):
    return pl.pallas_call(
        _final_body,
        grid=(GRID,),
        in_specs=[_row_spec] * 4 + [_w_spec, _b_spec],
        out_specs=_row_spec,
        out_shape=_nc_shape,
    )(st, s, t, ea, wls, bs2)


@functools.lru_cache(maxsize=1)
def _build_segsum():
    @functools.partial(
        pl.kernel,
        out_type=jax.ShapeDtypeStruct((2, N_ACC, C), jnp.float32),
        mesh=plsc.VectorSubcoreMesh(core_axis_name="c", subcore_axis_name="s",
                                    num_cores=2, num_subcores=NS),
        scratch_types=[
            pltpu.VMEM((IB, CHUNK), jnp.int32),
            pltpu.VMEM((IB, CHUNK), jnp.int32),
            pltpu.VMEM((CHUNK, C), jnp.float32),
            pltpu.VMEM((CHUNK, C), jnp.float32),
            pltpu.VMEM_SHARED((N_ACC, C), jnp.float32),
            pltpu.SemaphoreType.DMA,
        ],
    )
    def _segsum(p_hbm, ps_hbm, z_hbm, ct_hbm, cf_hbm, out,
                ct_v, cf_v, rows0, rows1, acc, sem):
        cid = lax.axis_index("c")
        sid = lax.axis_index("s")
        r0 = sid * ROWS_PER_TILE
        pltpu.sync_copy(z_hbm.at[pl.ds(r0, ROWS_PER_TILE)],
                        acc.at[pl.ds(r0, ROWS_PER_TILE)])
        plsc.subcore_barrier()

        bufs = (rows0, rows1)

        def run(tbl):
            def block(j, carry):
                pltpu.sync_copy(ct_hbm.at[sid, pl.ds(j * IB, IB)], ct_v)
                pltpu.sync_copy(cf_hbm.at[sid, pl.ds(j * IB, IB)], cf_v)
                desc = pltpu.async_copy(tbl.at[ct_v.at[0]], bufs[0], sem)
                for k in range(IB):
                    desc.wait()
                    if k + 1 < IB:
                        desc = pltpu.async_copy(tbl.at[ct_v.at[k + 1]],
                                                bufs[(k + 1) % 2], sem)
                    pltpu.sync_copy(bufs[k % 2], acc.at[cf_v.at[k]], add=True)
                return carry
            lax.fori_loop(0, KB, block, 0)
            plsc.subcore_barrier()
            pltpu.sync_copy(acc.at[pl.ds(r0, ROWS_PER_TILE)],
                            out.at[cid, pl.ds(r0, ROWS_PER_TILE)])

        @pl.when(cid == 0)
        def _():
            run(p_hbm)

        @pl.when(cid == 1)
        def _():
            run(ps_hbm)

    return _segsum


def kernel(objects, connections, object_state_W, state_attention_W,
           linked_state_attention_W, attention_b, linked_state_W, state_b):
    obj = objects[0]
    u = connections[0, :, 0]
    v = connections[0, :, 1]
    gat = jnp.concatenate([v, u])
    sca = jnp.concatenate([u, v])
    pad = EDGES_PAD - EDGES
    gat = jnp.concatenate([gat, jnp.zeros((pad,), jnp.int32)])
    sca = jnp.concatenate([sca, jnp.full((pad,), TRASH_ROW, jnp.int32)])
    ct_idx = gat.reshape(NS, K, CHUNK)
    cf_idx = sca.reshape(NS, K, CHUNK)
    zeros = jnp.zeros((N_ACC, C), jnp.float32)

    bs2 = state_b.reshape(1, C)
    ba2 = attention_b.reshape(1, C)

    st, p, ps, ea = _tc_init(obj, object_state_W, state_attention_W,
                             linked_state_attention_W, bs2, ba2)
    for it in range(NUM_ITER):
        o = _build_segsum()(p, ps, zeros, ct_idx, cf_idx)
        s_pad = o[0]
        t_pad = o[1]
        if it < NUM_ITER - 1:
            st, p, ps, ea = _tc_mid(st, s_pad, t_pad, ea, linked_state_W,
                                    state_attention_W, linked_state_attention_W,
                                    bs2, ba2)
        else:
            st = _tc_final(st, s_pad, t_pad, ea, linked_state_W, bs2)
    return st[None]


# speedup vs baseline: 5.0353x; 1.0844x over previous
import functools

import jax
import jax.numpy as jnp
from jax import lax
from jax.experimental import pallas as pl
from jax.experimental.pallas import tpu as pltpu
from jax.experimental.pallas import tpu_sc as plsc

N = 10000
E = 320000
F = 128
C = 128
NUM_ITER = 2

NS = 16
CHUNK = 128
IB = 32
EDGES = 2 * E
K = IB * (-(-EDGES // (NS * CHUNK * IB)))
KB = K // IB
EDGES_PAD = NS * K * CHUNK
ROWS_PER_TILE = 640
N_ACC = NS * ROWS_PER_TILE
TRASH_ROW = N_ACC - 2

BLK = 2000
GRID = N // BLK


def _init_body(obj_ref, wos_ref, wsa_ref, wlsa_ref, bs_ref, ba_ref,
               st_ref, p_ref, ps_ref, ea_ref):
    x = obj_ref[...]
    st = jnp.tanh(jnp.dot(x, wos_ref[...], preferred_element_type=jnp.float32)
                  + bs_ref[...])
    a = jnp.dot(st, wsa_ref[...], preferred_element_type=jnp.float32)
    l = jnp.dot(st, wlsa_ref[...], preferred_element_type=jnp.float32)
    p = jnp.exp(l)
    st_ref[...] = st
    p_ref[...] = p
    ps_ref[...] = p * st
    ea_ref[...] = jnp.exp(a + ba_ref[...])


def _mid_body(st_ref, s_ref, t_ref, ea_ref, wls_ref, wsa_ref, wlsa_ref,
              bs_ref, ba_ref, nst_ref, p_ref, ps_ref, nea_ref):
    st = st_ref[...]
    ea = ea_ref[...]
    inv = 1.0 / (ea * s_ref[0] + 1.0)
    g = ea * inv * t_ref[0]
    nst = jnp.tanh(st * inv
                   + jnp.dot(g, wls_ref[...], preferred_element_type=jnp.float32)
                   + bs_ref[...])
    a = jnp.dot(nst, wsa_ref[...], preferred_element_type=jnp.float32)
    l = jnp.dot(nst, wlsa_ref[...], preferred_element_type=jnp.float32)
    p = jnp.exp(l)
    nst_ref[...] = nst
    p_ref[...] = p
    ps_ref[...] = p * nst
    nea_ref[...] = jnp.exp(a + ba_ref[...])


def _final_body(st_ref, s_ref, t_ref, ea_ref, wls_ref, bs_ref, out_ref):
    st = st_ref[...]
    ea = ea_ref[...]
    inv = 1.0 / (ea * s_ref[0] + 1.0)
    g = ea * inv * t_ref[0]
    out_ref[...] = jnp.tanh(
        st * inv
        + jnp.dot(g, wls_ref[...], preferred_element_type=jnp.float32)
        + bs_ref[...])


_row_spec = pl.BlockSpec((BLK, C), lambda i: (i, 0))
_s_spec = pl.BlockSpec((1, BLK, C), lambda i: (0, i, 0))
_t_spec = pl.BlockSpec((1, BLK, C), lambda i: (1, i, 0))
_w_spec = pl.BlockSpec((C, C), lambda i: (0, 0))
_b_spec = pl.BlockSpec((1, C), lambda i: (0, 0))
_nc_shape = jax.ShapeDtypeStruct((N, C), jnp.float32)


def _tc_init(obj, wos, wsa, wlsa, bs2, ba2):
    return pl.pallas_call(
        _init_body,
        grid=(GRID,),
        in_specs=[_row_spec, _w_spec, _w_spec, _w_spec, _b_spec, _b_spec],
        out_specs=[_row_spec] * 4,
        out_shape=[_nc_shape] * 4,
    )(obj, wos, wsa, wlsa, bs2, ba2)


def _tc_mid(st, s, t, ea, wls, wsa, wlsa, bs2, ba2):
    return pl.pallas_call(
        _mid_body,
        grid=(GRID,),
        in_specs=[_row_spec, _s_spec, _t_spec, _row_spec]
        + [_w_spec] * 3 + [_b_spec] * 2,
        out_specs=[_row_spec] * 4,
        out_shape=[_nc_shape] * 4,
    )(st, s, t, ea, wls, wsa, wlsa, bs2, ba2)


def _tc_final(st, s, t, ea, wls, bs2):
    return pl.pallas_call(
        _final_body,
        grid=(GRID,),
        in_specs=[_row_spec, _s_spec, _t_spec, _row_spec, _w_spec, _b_spec],
        out_specs=_row_spec,
        out_shape=_nc_shape,
    )(st, s, t, ea, wls, bs2)


@functools.lru_cache(maxsize=1)
def _build_segsum():
    @functools.partial(
        pl.kernel,
        out_type=jax.ShapeDtypeStruct((2, N_ACC, C), jnp.float32),
        mesh=plsc.VectorSubcoreMesh(core_axis_name="c", subcore_axis_name="s",
                                    num_cores=2, num_subcores=NS),
        scratch_types=[
            pltpu.VMEM((IB, 2, CHUNK), jnp.int32),
            pltpu.VMEM((CHUNK, C), jnp.float32),
            pltpu.VMEM((CHUNK, C), jnp.float32),
            pltpu.VMEM_SHARED((N_ACC, C), jnp.float32),
            pltpu.SemaphoreType.DMA,
        ],
    )
    def _segsum(p_hbm, ps_hbm, z_hbm, idx_hbm, out,
                idx_v, rows0, rows1, acc, sem):
        cid = lax.axis_index("c")
        sid = lax.axis_index("s")
        r0 = sid * ROWS_PER_TILE
        pltpu.sync_copy(z_hbm.at[pl.ds(r0, ROWS_PER_TILE)],
                        acc.at[pl.ds(r0, ROWS_PER_TILE)])
        plsc.subcore_barrier()

        bufs = (rows0, rows1)

        def run(tbl):
            def block(j, carry):
                pltpu.sync_copy(idx_hbm.at[sid, pl.ds(j * IB, IB)], idx_v)
                desc = pltpu.async_copy(tbl.at[idx_v.at[0, 0]], bufs[0], sem)
                for k in range(IB):
                    desc.wait()
                    if k + 1 < IB:
                        desc = pltpu.async_copy(tbl.at[idx_v.at[k + 1, 0]],
                                                bufs[(k + 1) % 2], sem)
                    pltpu.sync_copy(bufs[k % 2], acc.at[idx_v.at[k, 1]],
                                    add=True)
                return carry
            lax.fori_loop(0, KB, block, 0)
            plsc.subcore_barrier()
            pltpu.sync_copy(acc.at[pl.ds(r0, ROWS_PER_TILE)],
                            out.at[cid, pl.ds(r0, ROWS_PER_TILE)])

        @pl.when(cid == 0)
        def _():
            run(p_hbm)

        @pl.when(cid == 1)
        def _():
            run(ps_hbm)

    return _segsum


def kernel(objects, connections, object_state_W, state_attention_W,
           linked_state_attention_W, attention_b, linked_state_W, state_b):
    obj = objects[0]
    u = connections[0, :, 0]
    v = connections[0, :, 1]
    gat = jnp.concatenate([v, u])
    sca = jnp.concatenate([u, v])
    pad = EDGES_PAD - EDGES
    gat = jnp.concatenate([gat, jnp.zeros((pad,), jnp.int32)])
    sca = jnp.concatenate([sca, jnp.full((pad,), TRASH_ROW, jnp.int32)])
    idx = jnp.stack([gat.reshape(NS, K, CHUNK),
                     sca.reshape(NS, K, CHUNK)], axis=2)
    zeros = jnp.zeros((N_ACC, C), jnp.float32)

    bs2 = state_b.reshape(1, C)
    ba2 = attention_b.reshape(1, C)

    st, p, ps, ea = _tc_init(obj, object_state_W, state_attention_W,
                             linked_state_attention_W, bs2, ba2)
    for it in range(NUM_ITER):
        o = _build_segsum()(p, ps, zeros, idx)
        if it < NUM_ITER - 1:
            st, p, ps, ea = _tc_mid(st, o, o, ea, linked_state_W,
                                    state_attention_W, linked_state_attention_W,
                                    bs2, ba2)
        else:
            st = _tc_final(st, o, o, ea, linked_state_W, bs2)
    return st[None]
